# baseline (speedup 1.0000x reference)
import functools

import jax
import jax.numpy as jnp
from jax import lax
from jax.experimental import pallas as pl
from jax.experimental.pallas import tpu as pltpu
from jax.experimental.pallas import tpu_sc as plsc

EMB_DIM = 32
NUM_HP_BINS = 256

NC = 2
NS = 16
NW = NC * NS

TROW = EMB_DIM + 1


def _make_kernel(b, u, n_utype):
  rows_w = b // NW
  n_look = rows_w * u
  n_bg = rows_w // 16
  mesh = plsc.VectorSubcoreMesh(
      core_axis_name="c", subcore_axis_name="s", num_cores=NC, num_subcores=NS
  )

  @functools.partial(
      pl.kernel,
      out_type=jax.ShapeDtypeStruct((u, 2 * EMB_DIM, b), jnp.float32),
      mesh=mesh,
      scratch_types=[
          pltpu.VMEM((n_utype * TROW,), jnp.float32),
          pltpu.VMEM((NUM_HP_BINS * TROW,), jnp.float32),
          pltpu.VMEM((n_look,), jnp.int32),
          pltpu.VMEM((n_look,), jnp.float32),
          pltpu.VMEM((2 * EMB_DIM, 128), jnp.float32),
          pltpu.VMEM((2 * EMB_DIM, 128), jnp.float32),
          pltpu.SemaphoreType.DMA,
          pltpu.SemaphoreType.DMA,
      ],
      compiler_params=pltpu.CompilerParams(needs_layout_passes=False),
  )
  def emb_kernel(ut_hbm, hp_hbm, utab_hbm, htab_hbm, out_hbm,
                 utab_v, htab_v, uidx_v, hp_v, cba, cbb, sema, semb):
    wid = lax.axis_index("s") * NC + lax.axis_index("c")
    kbase = wid * n_look
    b0 = wid * rows_w
    cbs = (cba, cbb)
    sems = (sema, semb)

    pltpu.sync_copy(utab_hbm, utab_v)
    pltpu.sync_copy(htab_hbm, htab_v)
    pltpu.sync_copy(ut_hbm.at[pl.ds(kbase, n_look)], uidx_v)
    pltpu.sync_copy(hp_hbm.at[pl.ds(kbase, n_look)], hp_v)

    l200 = lax.iota(jnp.int32, 16) * u

    def make_unit(cb):
      def unit_body(ui):
        @plsc.parallel_loop(0, n_bg, unroll=8)
        def bg_body(g):
          idxv = l200 + (g * (16 * u) + ui)
          uu = plsc.load_gather(uidx_v, [idxv]) * TROW
          hpv = plsc.load_gather(hp_v, [idxv])
          hh = (hpv * float(NUM_HP_BINS - 1)).astype(jnp.int32) * TROW
          for c in range(EMB_DIM):
            vu = plsc.load_gather(utab_v, [uu + c])
            cb[c, pl.ds(g * 16, 16)] = vu
            vh = plsc.load_gather(htab_v, [hh + c])
            cb[EMB_DIM + c, pl.ds(g * 16, 16)] = vh
      return unit_body

    def fire(ui, par):
      return pltpu.async_copy(
          cbs[par], out_hbm.at[ui, :, pl.ds(b0, 128)], sems[par])

    make_unit(cbs[0])(0)
    cp0 = fire(0, 0)
    make_unit(cbs[1])(1)
    cp1 = fire(1, 1)

    def unit_pair(p, _):
      ui = 2 * p + 2
      cp0.wait()
      make_unit(cbs[0])(ui)
      fire(ui, 0)
      cp1.wait()
      make_unit(cbs[1])(ui + 1)
      fire(ui + 1, 1)
      return ()

    lax.fori_loop(0, (u - 2) // 2, unit_pair, ())
    cp0.wait()
    cp1.wait()

  return emb_kernel


def kernel(utype, hp, utype_table, hp_table):
  b, u = utype.shape
  n_total = b * u
  n_utype = utype_table.shape[0]
  utab_p = jnp.pad(utype_table, ((0, 0), (0, TROW - EMB_DIM))).reshape(-1)
  htab_p = jnp.pad(hp_table, ((0, 0), (0, TROW - EMB_DIM))).reshape(-1)
  out_t = _make_kernel(b, u, n_utype)(
      utype.reshape(n_total).astype(jnp.int32), hp.reshape(n_total),
      utab_p, htab_p)
  return jnp.transpose(out_t, (2, 0, 1))


# speedup vs baseline: 1.4813x; 1.4813x over previous
import functools

import jax
import jax.numpy as jnp
from jax import lax
from jax.experimental import pallas as pl
from jax.experimental.pallas import tpu as pltpu
from jax.experimental.pallas import tpu_sc as plsc

EMB_DIM = 32
NUM_HP_BINS = 256

NC = 2
NS = 16
NW = NC * NS

TROW = EMB_DIM + 1


def _make_kernel(b, u, n_utype):
  rows_w = b // NW
  n_look = rows_w * u
  n_bg = rows_w // 16
  mesh = plsc.VectorSubcoreMesh(
      core_axis_name="c", subcore_axis_name="s", num_cores=NC, num_subcores=NS
  )

  @functools.partial(
      pl.kernel,
      out_type=jax.ShapeDtypeStruct((u, 2 * EMB_DIM, b), jnp.float32),
      mesh=mesh,
      scratch_types=[
          pltpu.VMEM((n_utype * TROW,), jnp.float32),
          pltpu.VMEM((NUM_HP_BINS * TROW,), jnp.float32),
          pltpu.VMEM((n_look,), jnp.int32),
          pltpu.VMEM((n_look,), jnp.float32),
          pltpu.VMEM((2 * EMB_DIM, 128), jnp.float32),
          pltpu.VMEM((2 * EMB_DIM, 128), jnp.float32),
          pltpu.SemaphoreType.DMA,
          pltpu.SemaphoreType.DMA,
      ],
      compiler_params=pltpu.CompilerParams(needs_layout_passes=False),
  )
  def emb_kernel(ut_hbm, hp_hbm, utab_hbm, htab_hbm, out_hbm,
                 utab_v, htab_v, uidx_v, hp_v, cba, cbb, sema, semb):
    wid = lax.axis_index("s") * NC + lax.axis_index("c")
    kbase = wid * n_look
    b0 = wid * rows_w
    cbs = (cba, cbb)
    sems = (sema, semb)

    pltpu.sync_copy(utab_hbm, utab_v)
    pltpu.sync_copy(htab_hbm, htab_v)
    pltpu.sync_copy(ut_hbm.at[pl.ds(kbase, n_look)], uidx_v)
    pltpu.sync_copy(hp_hbm.at[pl.ds(kbase, n_look)], hp_v)

    l200 = lax.iota(jnp.int32, 16) * u

    def make_unit(cb):
      def unit_body(ui):
        @plsc.parallel_loop(0, n_bg, unroll=4)
        def bg_body(g):
          idxv = l200 + (g * (16 * u) + ui)
          uu = plsc.load_gather(uidx_v, [idxv]) * TROW
          hpv = plsc.load_gather(hp_v, [idxv])
          hh = (hpv * float(NUM_HP_BINS - 1)).astype(jnp.int32) * TROW
          for c in range(EMB_DIM):
            vu = plsc.load_gather(utab_v, [uu + c])
            cb[c, pl.ds(g * 16, 16)] = vu
            vh = plsc.load_gather(htab_v, [hh + c])
            cb[EMB_DIM + c, pl.ds(g * 16, 16)] = vh
      return unit_body

    def fire(ui, par):
      return pltpu.async_copy(
          cbs[par], out_hbm.at[ui, :, pl.ds(b0, 128)], sems[par])

    make_unit(cbs[0])(0)
    cp0 = fire(0, 0)
    make_unit(cbs[1])(1)
    cp1 = fire(1, 1)

    def unit_pair(p, _):
      ui = 2 * p + 2
      cp0.wait()
      make_unit(cbs[0])(ui)
      fire(ui, 0)
      cp1.wait()
      make_unit(cbs[1])(ui + 1)
      fire(ui + 1, 1)
      return ()

    lax.fori_loop(0, (u - 2) // 2, unit_pair, ())
    cp0.wait()
    cp1.wait()

  return emb_kernel


def kernel(utype, hp, utype_table, hp_table):
  b, u = utype.shape
  n_total = b * u
  n_utype = utype_table.shape[0]
  utab_p = jnp.pad(utype_table, ((0, 0), (0, TROW - EMB_DIM))).reshape(-1)
  htab_p = jnp.pad(hp_table, ((0, 0), (0, TROW - EMB_DIM))).reshape(-1)
  out_t = _make_kernel(b, u, n_utype)(
      utype.reshape(n_total).astype(jnp.int32), hp.reshape(n_total),
      utab_p, htab_p)
  return jnp.transpose(out_t, (2, 0, 1))
